# baseline (speedup 1.0000x reference)
import functools

import jax
import jax.numpy as jnp
from jax import lax
from jax.experimental import pallas as pl
from jax.experimental.pallas import tpu as pltpu
from jax.experimental.pallas import tpu_sc as plsc

DIM = 768
LANES = 16
CHUNK = 32


@functools.lru_cache(maxsize=None)
def _build_sc_kernel(n_rows: int):
    info = plsc.get_sparse_core_info()
    nw = info.num_cores * info.num_subcores
    rows_per_w = n_rows // nw
    n_chunks = rows_per_w // CHUNK
    assert rows_per_w * nw == n_rows and n_chunks * CHUNK == rows_per_w
    assert n_chunks >= 8 and (n_chunks - 8) % 6 == 0

    mesh = plsc.VectorSubcoreMesh(core_axis_name="c", subcore_axis_name="s")

    @functools.partial(
        pl.kernel,
        mesh=mesh,
        out_type=jax.ShapeDtypeStruct((n_rows, DIM), jnp.float32),
        scratch_types=[
            [pltpu.VMEM((CHUNK,), jnp.int32) for _ in range(2)],
            [pltpu.VMEM((CHUNK, DIM), jnp.float32) for _ in range(3)],
            [pltpu.VMEM((CHUNK, DIM), jnp.float32) for _ in range(2)],
            [pltpu.SemaphoreType.DMA for _ in range(2)],
            [pltpu.SemaphoreType.DMA for _ in range(3)],
            [pltpu.SemaphoreType.DMA for _ in range(2)],
            [pltpu.SemaphoreType.DMA for _ in range(3)],
        ],
    )
    def k(x_hbm, idx_hbm, tab_hbm, out_hbm,
          idx_v, pe_v, x_v, sem_i, sem_g, sem_x, sem_o):
        c = lax.axis_index("c")
        s = lax.axis_index("s")
        wid = s * info.num_cores + c
        base = wid * rows_per_w

        def row0(g):
            return base + g * CHUNK

        def issue_idx(g, sl):
            pltpu.async_copy(idx_hbm.at[pl.ds(row0(g), CHUNK)], idx_v[sl],
                             sem_i[sl])

        def wait_idx(sl):
            pltpu.make_async_copy(idx_hbm.at[pl.ds(base, CHUNK)], idx_v[sl],
                                  sem_i[sl]).wait()

        def issue_gather(isl, sl):
            pltpu.async_copy(tab_hbm.at[idx_v[isl]], pe_v[sl], sem_g[sl])

        def wait_gather(isl, sl):
            pltpu.make_async_copy(tab_hbm.at[idx_v[isl]], pe_v[sl],
                                  sem_g[sl]).wait()

        def issue_x(g, sl):
            pltpu.async_copy(x_hbm.at[pl.ds(row0(g), CHUNK)], x_v[sl],
                             sem_x[sl])

        def wait_x(sl):
            pltpu.make_async_copy(x_hbm.at[pl.ds(base, CHUNK)], x_v[sl],
                                  sem_x[sl]).wait()

        def issue_out(g, sl):
            pltpu.async_copy(pe_v[sl], out_hbm.at[pl.ds(row0(g), CHUNK)],
                             sem_o[sl])

        def wait_out(sl):
            pltpu.make_async_copy(pe_v[sl], out_hbm.at[pl.ds(base, CHUNK)],
                                  sem_o[sl]).wait()

        def compute(pp, px):
            pb, xb = pe_v[pp], x_v[px]

            def row_body(r, carry):
                for j in range(DIM // LANES):
                    sl = pl.ds(j * LANES, LANES)
                    pb[r, sl] = pb[r, sl] + xb[r, sl]
                return carry

            lax.fori_loop(0, CHUNK, row_body, 0)

        def step(g, gmod, do_owait=True, do_loads=True, do_idx=True):
            sp, sx, si = gmod % 3, gmod % 2, gmod % 2
            pp, px = (gmod - 1) % 3, (gmod - 1) % 2
            if do_loads:
                if do_owait:
                    wait_out(sp)
                wait_idx(si)
                issue_gather(si, sp)
                issue_x(g, sx)
            wait_gather((gmod - 1) % 2, pp)
            wait_x(px)
            if do_idx:
                issue_idx(g + 1, (gmod + 1) % 2)
            compute(pp, px)
            issue_out(g - 1, pp)

        pltpu.sync_copy(idx_hbm.at[pl.ds(row0(0), CHUNK)], idx_v[0])
        issue_gather(0, 0)
        issue_x(0, 0)
        issue_idx(1, 1)

        step(1, 1, do_owait=False)
        step(2, 2, do_owait=False)

        def six_body(t, carry):
            gg = 3 + 6 * t
            for kk in range(6):
                step(gg + kk, (3 + kk) % 6)
            return carry

        lax.fori_loop(0, (n_chunks - 8) // 6, six_body, 0)

        for g in range(n_chunks - 5, n_chunks - 1):
            step(g, g % 6)
        step(n_chunks - 1, (n_chunks - 1) % 6, do_idx=False)
        step(n_chunks, n_chunks % 6, do_loads=False, do_idx=False)

        wait_out(0)
        wait_out(1)
        wait_out(2)

    return k


def kernel(x, aa_idx, pos_enc):
    b, one, l, d = x.shape
    n = b * l
    xf = x.reshape(n, d)
    idx = aa_idx.reshape(n).astype(jnp.int32)
    out = _build_sc_kernel(n)(xf, idx, pos_enc)
    return out.reshape(b, one, l, d)


# speedup vs baseline: 1.0376x; 1.0376x over previous
import functools

import jax
import jax.numpy as jnp
from jax import lax
from jax.experimental import pallas as pl
from jax.experimental.pallas import tpu as pltpu
from jax.experimental.pallas import tpu_sc as plsc

DIM = 768
LANES = 16
CHUNK = 32


@functools.lru_cache(maxsize=None)
def _build_sc_kernel(n_rows: int):
    info = plsc.get_sparse_core_info()
    nw = info.num_cores * info.num_subcores
    rows_per_w = n_rows // nw
    n_chunks = rows_per_w // CHUNK
    assert rows_per_w * nw == n_rows and n_chunks * CHUNK == rows_per_w
    assert n_chunks >= 6 and n_chunks % 2 == 0

    mesh = plsc.VectorSubcoreMesh(core_axis_name="c", subcore_axis_name="s")

    @functools.partial(
        pl.kernel,
        mesh=mesh,
        out_type=jax.ShapeDtypeStruct((n_rows, DIM), jnp.float32),
        scratch_types=[
            [pltpu.VMEM((CHUNK,), jnp.int32) for _ in range(2)],
            [pltpu.VMEM((CHUNK, DIM), jnp.float32) for _ in range(2)],
            [pltpu.VMEM((CHUNK, DIM), jnp.float32) for _ in range(2)],
            [pltpu.SemaphoreType.DMA for _ in range(2)],
            [pltpu.SemaphoreType.DMA for _ in range(2)],
            [pltpu.SemaphoreType.DMA for _ in range(2)],
            [pltpu.SemaphoreType.DMA for _ in range(2)],
        ],
    )
    def k(x_hbm, idx_hbm, tab_hbm, out_hbm,
          idx_v, pe_v, x_v, sem_i, sem_g, sem_x, sem_o):
        c = lax.axis_index("c")
        s = lax.axis_index("s")
        wid = s * info.num_cores + c
        base = wid * rows_per_w

        def row0(g):
            return base + g * CHUNK

        def issue_idx(g, sl):
            pltpu.async_copy(idx_hbm.at[pl.ds(row0(g), CHUNK)], idx_v[sl],
                             sem_i[sl])

        def wait_idx(sl):
            pltpu.make_async_copy(idx_hbm.at[pl.ds(base, CHUNK)], idx_v[sl],
                                  sem_i[sl]).wait()

        def issue_gather(sl):
            pltpu.async_copy(tab_hbm.at[idx_v[sl]], pe_v[sl], sem_g[sl])

        def wait_gather(sl):
            pltpu.make_async_copy(tab_hbm.at[idx_v[sl]], pe_v[sl],
                                  sem_g[sl]).wait()

        def issue_x(g, sl):
            pltpu.async_copy(x_hbm.at[pl.ds(row0(g), CHUNK)], x_v[sl],
                             sem_x[sl])

        def wait_x(sl):
            pltpu.make_async_copy(x_hbm.at[pl.ds(base, CHUNK)], x_v[sl],
                                  sem_x[sl]).wait()

        def issue_out(g, sl):
            pltpu.async_copy(x_v[sl], out_hbm.at[pl.ds(row0(g), CHUNK)],
                             sem_o[sl])

        def wait_out(sl):
            pltpu.make_async_copy(x_v[sl], out_hbm.at[pl.ds(base, CHUNK)],
                                  sem_o[sl]).wait()

        def compute(p):
            xb, pb = x_v[p], pe_v[p]

            def row_body(r, carry):
                for j in range(DIM // LANES):
                    sl = pl.ds(j * LANES, LANES)
                    xb[r, sl] = xb[r, sl] + pb[r, sl]
                return carry

            lax.fori_loop(0, CHUNK, row_body, 0)

        def step(g, sl, do_owait, do_loads, do_idx):
            p = 1 - sl
            if do_loads:
                if do_owait:
                    wait_out(sl)
                wait_idx(sl)
                issue_gather(sl)
                issue_x(g, sl)
            wait_gather(p)
            wait_x(p)
            if do_idx:
                issue_idx(g + 1, p)
            compute(p)
            issue_out(g - 1, p)

        pltpu.sync_copy(idx_hbm.at[pl.ds(row0(0), CHUNK)], idx_v[0])
        issue_gather(0)
        issue_x(0, 0)
        issue_idx(1, 1)

        step(1, 1, False, True, True)
        step(2, 0, True, True, True)

        def pair_body(t, carry):
            gg = 3 + 2 * t
            step(gg, 1, True, True, True)
            step(gg + 1, 0, True, True, True)
            return carry

        lax.fori_loop(0, (n_chunks - 4) // 2, pair_body, 0)

        step(n_chunks - 1, 1, True, True, False)
        step(n_chunks, 0, False, False, False)

        wait_out(0)
        wait_out(1)

    return k


def kernel(x, aa_idx, pos_enc):
    b, one, l, d = x.shape
    n = b * l
    xf = x.reshape(n, d)
    idx = aa_idx.reshape(n).astype(jnp.int32)
    out = _build_sc_kernel(n)(xf, idx, pos_enc)
    return out.reshape(b, one, l, d)
